# baseline (speedup 1.0000x reference)
import functools

import jax
import jax.numpy as jnp
from jax import lax
from jax.experimental import pallas as pl
from jax.experimental.pallas import tpu as pltpu
from jax.experimental.pallas import tpu_sc as plsc

LANES = 16


def _sc_worker_count() -> tuple[int, int]:
  try:
    info = plsc.get_sparse_core_info()
    return info.num_cores, info.num_subcores
  except Exception:
    return 2, 16


@functools.lru_cache(maxsize=None)
def _build_pool(batch: int, hist: int, dim: int):
  nc, ns = _sc_worker_count()
  nw = nc * ns
  assert batch % nw == 0
  bpw = batch // nw
  nbuf = 4
  chunk = 64
  assert bpw % chunk == 0 and chunk % nbuf == 0
  nch = bpw // chunk
  ngrp = chunk // nbuf - 1
  split = 128
  rest = hist - split
  assert 0 < rest <= 128 and hist % 8 == 0 and dim % LANES == 0
  nd = dim // LANES
  inv = jnp.float32(1.0 / hist)

  mesh = plsc.VectorSubcoreMesh(core_axis_name="c", subcore_axis_name="s")

  @functools.partial(
      pl.kernel,
      out_type=jax.ShapeDtypeStruct((batch, dim), jnp.float32),
      mesh=mesh,
      scratch_types=[
          pltpu.VMEM((chunk * hist,), jnp.int32),
          pltpu.VMEM((nbuf, hist, dim), jnp.float32),
          pltpu.VMEM((bpw, dim), jnp.float32),
          pltpu.SemaphoreType.DMA((nbuf,)),
      ],
      compiler_params=pltpu.CompilerParams(use_tc_tiling_on_sc=False),
  )
  def pool(x_hbm, table_hbm, out_hbm, idx_v, rows_v, acc_v, sem):
    wid = lax.axis_index("s") * nc + lax.axis_index("c")
    row0 = wid * bpw

    def issue(crow, slot):
      off = crow * hist
      pltpu.async_copy(
          table_hbm.at[idx_v.at[pl.ds(off, split)]],
          rows_v.at[slot, pl.ds(0, split)],
          sem.at[slot],
      )
      pltpu.async_copy(
          table_hbm.at[idx_v.at[pl.ds(off + split, rest)]],
          rows_v.at[slot, pl.ds(split, rest)],
          sem.at[slot],
      )

    def wait(slot):
      pltpu.make_async_copy(
          table_hbm.at[pl.ds(0, split)],
          rows_v.at[slot, pl.ds(0, split)],
          sem.at[slot],
      ).wait()
      pltpu.make_async_copy(
          table_hbm.at[pl.ds(0, rest)],
          rows_v.at[slot, pl.ds(split, rest)],
          sem.at[slot],
      ).wait()

    def reduce(slot, brow):
      r = rows_v.at[slot]

      def step(j, carry):
        return tuple(
            carry[d] + r[j, pl.ds(LANES * d, LANES)] for d in range(nd)
        )

      zeros = (jnp.zeros((LANES,), jnp.float32),) * nd
      acc = pl.loop(0, hist, init_carry=zeros, unroll=8)(step)
      for d in range(nd):
        acc_v[brow, pl.ds(LANES * d, LANES)] = acc[d] * inv

    def chunk_body(c):
      base = c * chunk
      pltpu.sync_copy(
          x_hbm.at[pl.ds((row0 + base) * hist, chunk * hist)], idx_v
      )
      for k in range(nbuf):
        issue(k, k)

      def grp(g):
        for k in range(nbuf):
          j = g * nbuf + k
          wait(k)
          reduce(k, base + j)
          issue(j + nbuf, k)

      pl.loop(0, ngrp)(grp)
      for k in range(nbuf):
        wait(k)
        reduce(k, base + (ngrp * nbuf + k))

    pl.loop(0, nch)(chunk_body)
    pltpu.sync_copy(acc_v, out_hbm.at[pl.ds(row0, bpw)])

  return pool


_PACK_B = 768
_S = 499968
_R = 500032


def _pack_body(t1_ref, t2_ref, o_ref):
  o_ref[:, 0:64] = t1_ref[...].T
  o_ref[:, 64:128] = t2_ref[...].T


@functools.lru_cache(maxsize=None)
def _build_pack(vocab: int, dim: int):
  assert dim == 64 and vocab == 2 * _S + 64
  grid = (_R + _PACK_B - 1) // _PACK_B
  off = _S // _PACK_B
  return pl.pallas_call(
      _pack_body,
      grid=(grid,),
      in_specs=[
          pl.BlockSpec((dim, _PACK_B), lambda i: (0, i)),
          pl.BlockSpec((dim, _PACK_B), lambda i: (0, i + off)),
      ],
      out_specs=pl.BlockSpec((_PACK_B, 128), lambda i: (i, 0)),
      out_shape=jax.ShapeDtypeStruct((_R, 128), jnp.float32),
  )


def _mm_body(p_ref, w_ref, b_ref, o_ref):
  o_ref[...] = (
      jnp.dot(p_ref[...], w_ref[...], preferred_element_type=jnp.float32)
      + b_ref[...]
  )


@functools.lru_cache(maxsize=None)
def _build_linear(batch: int, dim: int, odim: int):
  bm = 2048
  assert batch % bm == 0
  return pl.pallas_call(
      _mm_body,
      grid=(batch // bm,),
      in_specs=[
          pl.BlockSpec((bm, dim), lambda i: (i, 0)),
          pl.BlockSpec((dim, odim), lambda i: (0, 0)),
          pl.BlockSpec((1, odim), lambda i: (0, 0)),
      ],
      out_specs=pl.BlockSpec((bm, odim), lambda i: (i, 0)),
      out_shape=jax.ShapeDtypeStruct((batch, odim), jnp.float32),
  )


def kernel(x, table, W, b):
  batch, hist = x.shape
  vocab, dim = table.shape
  odim = W.shape[1]
  x_flat = jnp.asarray(x, jnp.int32).reshape(batch * hist)
  x_flat = jnp.where(x_flat < _R, 2 * x_flat, 2 * x_flat - (2 * _S - 1))
  table_lin = _build_pack(vocab, dim)(table.T, table.T).reshape(2 * _R, dim)
  pooled = _build_pool(batch, hist, dim)(x_flat, table_lin)
  return _build_linear(batch, dim, odim)(pooled, W, b.reshape(1, odim))


# speedup vs baseline: 1.4300x; 1.4300x over previous
import functools

import jax
import jax.numpy as jnp
from jax import lax
from jax.experimental import pallas as pl
from jax.experimental.pallas import tpu as pltpu
from jax.experimental.pallas import tpu_sc as plsc

LANES = 16


def _sc_worker_count() -> tuple[int, int]:
  try:
    info = plsc.get_sparse_core_info()
    return info.num_cores, info.num_subcores
  except Exception:
    return 2, 16


@functools.lru_cache(maxsize=None)
def _build_pool(batch: int, hist: int, dim: int):
  nc, ns = _sc_worker_count()
  nw = nc * ns
  assert batch % nw == 0
  bpw = batch // nw
  nbuf = 4
  chunk = 64
  assert bpw % chunk == 0 and chunk % nbuf == 0
  nch = bpw // chunk
  ngrp = chunk // nbuf - 1
  split = 128
  rest = hist - split
  assert 0 < rest <= 128 and hist % 8 == 0 and dim % LANES == 0
  nd = dim // LANES
  inv = jnp.float32(1.0 / hist)

  mesh = plsc.VectorSubcoreMesh(core_axis_name="c", subcore_axis_name="s")

  @functools.partial(
      pl.kernel,
      out_type=jax.ShapeDtypeStruct((batch, dim), jnp.float32),
      mesh=mesh,
      scratch_types=[
          pltpu.VMEM((chunk * hist,), jnp.int32),
          pltpu.VMEM((nbuf, hist, dim), jnp.float32),
          pltpu.VMEM((bpw, dim), jnp.float32),
          pltpu.SemaphoreType.DMA((nbuf,)),
      ],
      compiler_params=pltpu.CompilerParams(use_tc_tiling_on_sc=False),
  )
  def pool(x_hbm, table_hbm, out_hbm, idx_v, rows_v, acc_v, sem):
    wid = lax.axis_index("s") * nc + lax.axis_index("c")
    row0 = wid * bpw

    def issue(crow, slot):
      off = crow * hist
      pltpu.async_copy(
          table_hbm.at[idx_v.at[pl.ds(off, split)]],
          rows_v.at[slot, pl.ds(0, split)],
          sem.at[slot],
      )
      pltpu.async_copy(
          table_hbm.at[idx_v.at[pl.ds(off + split, rest)]],
          rows_v.at[slot, pl.ds(split, rest)],
          sem.at[slot],
      )

    def wait(slot):
      pltpu.make_async_copy(
          table_hbm.at[pl.ds(0, split)],
          rows_v.at[slot, pl.ds(0, split)],
          sem.at[slot],
      ).wait()
      pltpu.make_async_copy(
          table_hbm.at[pl.ds(0, rest)],
          rows_v.at[slot, pl.ds(split, rest)],
          sem.at[slot],
      ).wait()

    def reduce(slot, brow):
      r = rows_v.at[slot]

      def step(j, carry):
        return tuple(
            carry[d] + r[j, pl.ds(LANES * d, LANES)] for d in range(nd)
        )

      zeros = (jnp.zeros((LANES,), jnp.float32),) * nd
      acc = pl.loop(0, hist, init_carry=zeros, unroll=8)(step)
      for d in range(nd):
        acc_v[brow, pl.ds(LANES * d, LANES)] = acc[d] * inv

    def chunk_body(c):
      base = c * chunk
      pltpu.sync_copy(
          x_hbm.at[pl.ds((row0 + base) * hist, chunk * hist)], idx_v
      )
      for k in range(nbuf):
        issue(k, k)

      def grp(g):
        for k in range(nbuf):
          j = g * nbuf + k
          wait(k)
          reduce(k, base + j)
          issue(j + nbuf, k)

      pl.loop(0, ngrp)(grp)
      for k in range(nbuf):
        wait(k)
        reduce(k, base + (ngrp * nbuf + k))

    pl.loop(0, nch)(chunk_body)
    pltpu.sync_copy(acc_v, out_hbm.at[pl.ds(row0, bpw)])

  return pool


_PACK_B = 5376
_S = 499968
_R = 500032


def _pack_body(t1_ref, t2_ref, o_ref):
  eye = jnp.eye(64, dtype=jnp.float32)
  dn = (((0,), (0,)), ((), ()))
  o_ref[:, 0:64] = lax.dot_general(
      t1_ref[...], eye, dn, preferred_element_type=jnp.float32)
  o_ref[:, 64:128] = lax.dot_general(
      t2_ref[...], eye, dn, preferred_element_type=jnp.float32)


@functools.lru_cache(maxsize=None)
def _build_pack(vocab: int, dim: int):
  assert dim == 64 and vocab == 2 * _S + 64
  grid = (_R + _PACK_B - 1) // _PACK_B
  off = _S // _PACK_B
  return pl.pallas_call(
      _pack_body,
      grid=(grid,),
      in_specs=[
          pl.BlockSpec((dim, _PACK_B), lambda i: (0, i)),
          pl.BlockSpec((dim, _PACK_B), lambda i: (0, i + off)),
      ],
      out_specs=pl.BlockSpec((_PACK_B, 128), lambda i: (i, 0)),
      out_shape=jax.ShapeDtypeStruct((_R, 128), jnp.float32),
  )


def _mm_body(p_ref, w_ref, b_ref, o_ref):
  o_ref[...] = (
      jnp.dot(p_ref[...], w_ref[...], preferred_element_type=jnp.float32)
      + b_ref[...]
  )


@functools.lru_cache(maxsize=None)
def _build_linear(batch: int, dim: int, odim: int):
  bm = 2048
  assert batch % bm == 0
  return pl.pallas_call(
      _mm_body,
      grid=(batch // bm,),
      in_specs=[
          pl.BlockSpec((bm, dim), lambda i: (i, 0)),
          pl.BlockSpec((dim, odim), lambda i: (0, 0)),
          pl.BlockSpec((1, odim), lambda i: (0, 0)),
      ],
      out_specs=pl.BlockSpec((bm, odim), lambda i: (i, 0)),
      out_shape=jax.ShapeDtypeStruct((batch, odim), jnp.float32),
  )


def kernel(x, table, W, b):
  batch, hist = x.shape
  vocab, dim = table.shape
  odim = W.shape[1]
  x_flat = jnp.asarray(x, jnp.int32).reshape(batch * hist)
  x_flat = jnp.where(x_flat < _R, 2 * x_flat, 2 * x_flat - (2 * _S - 1))
  table_lin = _build_pack(vocab, dim)(table.T, table.T).reshape(2 * _R, dim)
  pooled = _build_pool(batch, hist, dim)(x_flat, table_lin)
  return _build_linear(batch, dim, odim)(pooled, W, b.reshape(1, odim))


# speedup vs baseline: 1.4956x; 1.0459x over previous
import functools

import jax
import jax.numpy as jnp
from jax import lax
from jax.experimental import pallas as pl
from jax.experimental.pallas import tpu as pltpu
from jax.experimental.pallas import tpu_sc as plsc

LANES = 16


def _sc_worker_count() -> tuple[int, int]:
  try:
    info = plsc.get_sparse_core_info()
    return info.num_cores, info.num_subcores
  except Exception:
    return 2, 16


@functools.lru_cache(maxsize=None)
def _build_pool(batch: int, hist: int, dim: int):
  nc, ns = _sc_worker_count()
  nw = nc * ns
  assert batch % nw == 0
  bpw = batch // nw
  nbuf = 4
  chunk = 64
  assert bpw % chunk == 0 and chunk % nbuf == 0
  nch = bpw // chunk
  ngrp = chunk // nbuf - 1
  split = 128
  rest = hist - split
  assert 0 < rest <= 128 and hist % 8 == 0 and dim % LANES == 0
  nd = dim // LANES
  inv = jnp.float32(1.0 / hist)

  mesh = plsc.VectorSubcoreMesh(core_axis_name="c", subcore_axis_name="s")

  @functools.partial(
      pl.kernel,
      out_type=jax.ShapeDtypeStruct((batch, dim), jnp.float32),
      mesh=mesh,
      scratch_types=[
          pltpu.VMEM((chunk * hist,), jnp.int32),
          pltpu.VMEM((nbuf, hist, dim), jnp.float32),
          pltpu.VMEM((bpw, dim), jnp.float32),
          pltpu.SemaphoreType.DMA((nbuf,)),
      ],
      compiler_params=pltpu.CompilerParams(use_tc_tiling_on_sc=False),
  )
  def pool(x_hbm, table_hbm, out_hbm, idx_v, rows_v, acc_v, sem):
    wid = lax.axis_index("s") * nc + lax.axis_index("c")
    row0 = wid * bpw

    def issue(crow, slot):
      off = crow * hist
      pltpu.async_copy(
          table_hbm.at[idx_v.at[pl.ds(off, split)]],
          rows_v.at[slot, pl.ds(0, split)],
          sem.at[slot],
      )
      pltpu.async_copy(
          table_hbm.at[idx_v.at[pl.ds(off + split, rest)]],
          rows_v.at[slot, pl.ds(split, rest)],
          sem.at[slot],
      )

    def wait(slot):
      pltpu.make_async_copy(
          table_hbm.at[pl.ds(0, split)],
          rows_v.at[slot, pl.ds(0, split)],
          sem.at[slot],
      ).wait()
      pltpu.make_async_copy(
          table_hbm.at[pl.ds(0, rest)],
          rows_v.at[slot, pl.ds(split, rest)],
          sem.at[slot],
      ).wait()

    def reduce(slot, brow):
      r = rows_v.at[slot]

      def step(j, carry):
        return tuple(
            carry[d] + r[j, pl.ds(LANES * d, LANES)] for d in range(nd)
        )

      zeros = (jnp.zeros((LANES,), jnp.float32),) * nd
      acc = pl.loop(0, hist, init_carry=zeros, unroll=8)(step)
      for d in range(nd):
        acc_v[brow, pl.ds(LANES * d, LANES)] = acc[d] * inv

    def chunk_body(c):
      base = c * chunk
      pltpu.sync_copy(
          x_hbm.at[pl.ds((row0 + base) * hist, chunk * hist)], idx_v
      )
      for k in range(nbuf):
        issue(k, k)

      def grp(g):
        for k in range(nbuf):
          j = g * nbuf + k
          wait(k)
          reduce(k, base + j)
          issue(j + nbuf, k)

      pl.loop(0, ngrp)(grp)
      for k in range(nbuf):
        wait(k)
        reduce(k, base + (ngrp * nbuf + k))

    pl.loop(0, nch)(chunk_body)
    pltpu.sync_copy(acc_v, out_hbm.at[pl.ds(row0, bpw)])

  return pool


_PACK_B = 16128
_S = 499968
_R = 500032


def _pack_body(t1_ref, t2_ref, o_ref):
  eye = jnp.eye(64, dtype=jnp.float32)
  dn = (((0,), (0,)), ((), ()))
  o_ref[:, 0:64] = lax.dot_general(
      t1_ref[...], eye, dn, preferred_element_type=jnp.float32)
  o_ref[:, 64:128] = lax.dot_general(
      t2_ref[...], eye, dn, preferred_element_type=jnp.float32)


@functools.lru_cache(maxsize=None)
def _build_pack(vocab: int, dim: int):
  assert dim == 64 and vocab == 2 * _S + 64
  grid = (_R + _PACK_B - 1) // _PACK_B
  off = _S // _PACK_B
  return pl.pallas_call(
      _pack_body,
      grid=(grid,),
      in_specs=[
          pl.BlockSpec((dim, _PACK_B), lambda i: (0, i)),
          pl.BlockSpec((dim, _PACK_B), lambda i: (0, i + off)),
      ],
      out_specs=pl.BlockSpec((_PACK_B, 128), lambda i: (i, 0)),
      out_shape=jax.ShapeDtypeStruct((_R, 128), jnp.float32),
  )


def _mm_body(p_ref, w_ref, b_ref, o_ref):
  o_ref[...] = (
      jnp.dot(p_ref[...], w_ref[...], preferred_element_type=jnp.float32)
      + b_ref[...]
  )


@functools.lru_cache(maxsize=None)
def _build_linear(batch: int, dim: int, odim: int):
  bm = 2048
  assert batch % bm == 0
  return pl.pallas_call(
      _mm_body,
      grid=(batch // bm,),
      in_specs=[
          pl.BlockSpec((bm, dim), lambda i: (i, 0)),
          pl.BlockSpec((dim, odim), lambda i: (0, 0)),
          pl.BlockSpec((1, odim), lambda i: (0, 0)),
      ],
      out_specs=pl.BlockSpec((bm, odim), lambda i: (i, 0)),
      out_shape=jax.ShapeDtypeStruct((batch, odim), jnp.float32),
  )


def kernel(x, table, W, b):
  batch, hist = x.shape
  vocab, dim = table.shape
  odim = W.shape[1]
  x_flat = jnp.asarray(x, jnp.int32).reshape(batch * hist)
  x_flat = jnp.where(x_flat < _R, 2 * x_flat, 2 * x_flat - (2 * _S - 1))
  table_lin = _build_pack(vocab, dim)(table.T, table.T).reshape(2 * _R, dim)
  pooled = _build_pool(batch, hist, dim)(x_flat, table_lin)
  return _build_linear(batch, dim, odim)(pooled, W, b.reshape(1, odim))


# speedup vs baseline: 1.5267x; 1.0208x over previous
import functools

import jax
import jax.numpy as jnp
from jax import lax
from jax.experimental import pallas as pl
from jax.experimental.pallas import tpu as pltpu
from jax.experimental.pallas import tpu_sc as plsc

LANES = 16


def _sc_worker_count() -> tuple[int, int]:
  try:
    info = plsc.get_sparse_core_info()
    return info.num_cores, info.num_subcores
  except Exception:
    return 2, 16


@functools.lru_cache(maxsize=None)
def _build_pool(batch: int, hist: int, dim: int):
  nc, ns = _sc_worker_count()
  nw = nc * ns
  assert batch % nw == 0
  bpw = batch // nw
  nbuf = 4
  chunk = 128
  assert bpw % chunk == 0 and chunk % nbuf == 0
  nch = bpw // chunk
  ngrp = chunk // nbuf - 1
  split = 128
  rest = hist - split
  assert 0 < rest <= 128 and hist % 8 == 0 and dim % LANES == 0
  nd = dim // LANES
  inv = jnp.float32(1.0 / hist)

  mesh = plsc.VectorSubcoreMesh(core_axis_name="c", subcore_axis_name="s")

  @functools.partial(
      pl.kernel,
      out_type=jax.ShapeDtypeStruct((batch, dim), jnp.float32),
      mesh=mesh,
      scratch_types=[
          pltpu.VMEM((chunk * hist,), jnp.int32),
          pltpu.VMEM((nbuf, hist, dim), jnp.float32),
          pltpu.VMEM((bpw, dim), jnp.float32),
          pltpu.SemaphoreType.DMA((nbuf,)),
      ],
      compiler_params=pltpu.CompilerParams(use_tc_tiling_on_sc=False),
  )
  def pool(x_hbm, table_hbm, out_hbm, idx_v, rows_v, acc_v, sem):
    wid = lax.axis_index("s") * nc + lax.axis_index("c")
    row0 = wid * bpw

    def issue(crow, slot):
      off = crow * hist
      pltpu.async_copy(
          table_hbm.at[idx_v.at[pl.ds(off, split)]],
          rows_v.at[slot, pl.ds(0, split)],
          sem.at[slot],
      )
      pltpu.async_copy(
          table_hbm.at[idx_v.at[pl.ds(off + split, rest)]],
          rows_v.at[slot, pl.ds(split, rest)],
          sem.at[slot],
      )

    def wait(slot):
      pltpu.make_async_copy(
          table_hbm.at[pl.ds(0, split)],
          rows_v.at[slot, pl.ds(0, split)],
          sem.at[slot],
      ).wait()
      pltpu.make_async_copy(
          table_hbm.at[pl.ds(0, rest)],
          rows_v.at[slot, pl.ds(split, rest)],
          sem.at[slot],
      ).wait()

    def reduce(slot, brow):
      r = rows_v.at[slot]

      def step(j, carry):
        return tuple(
            carry[d] + r[j, pl.ds(LANES * d, LANES)] for d in range(nd)
        )

      zeros = (jnp.zeros((LANES,), jnp.float32),) * nd
      acc = pl.loop(0, hist, init_carry=zeros, unroll=8)(step)
      for d in range(nd):
        acc_v[brow, pl.ds(LANES * d, LANES)] = acc[d] * inv

    def chunk_body(c):
      base = c * chunk
      pltpu.sync_copy(
          x_hbm.at[pl.ds((row0 + base) * hist, chunk * hist)], idx_v
      )
      for k in range(nbuf):
        issue(k, k)

      def grp(g):
        for k in range(nbuf):
          j = g * nbuf + k
          wait(k)
          reduce(k, base + j)
          issue(j + nbuf, k)

      pl.loop(0, ngrp)(grp)
      for k in range(nbuf):
        wait(k)
        reduce(k, base + (ngrp * nbuf + k))

    pl.loop(0, nch)(chunk_body)
    pltpu.sync_copy(acc_v, out_hbm.at[pl.ds(row0, bpw)])

  return pool


_PACK_B = 16128
_S = 499968
_R = 500032


def _pack_body(t1_ref, t2_ref, o_ref):
  eye = jnp.eye(64, dtype=jnp.float32)
  dn = (((0,), (0,)), ((), ()))
  o_ref[:, 0:64] = lax.dot_general(
      t1_ref[...], eye, dn, preferred_element_type=jnp.float32)
  o_ref[:, 64:128] = lax.dot_general(
      t2_ref[...], eye, dn, preferred_element_type=jnp.float32)


@functools.lru_cache(maxsize=None)
def _build_pack(vocab: int, dim: int):
  assert dim == 64 and vocab == 2 * _S + 64
  grid = (_R + _PACK_B - 1) // _PACK_B
  off = _S // _PACK_B
  return pl.pallas_call(
      _pack_body,
      grid=(grid,),
      in_specs=[
          pl.BlockSpec((dim, _PACK_B), lambda i: (0, i)),
          pl.BlockSpec((dim, _PACK_B), lambda i: (0, i + off)),
      ],
      out_specs=pl.BlockSpec((_PACK_B, 128), lambda i: (i, 0)),
      out_shape=jax.ShapeDtypeStruct((_R, 128), jnp.float32),
  )


def _mm_body(p_ref, w_ref, b_ref, o_ref):
  o_ref[...] = (
      jnp.dot(p_ref[...], w_ref[...], preferred_element_type=jnp.float32)
      + b_ref[...]
  )


@functools.lru_cache(maxsize=None)
def _build_linear(batch: int, dim: int, odim: int):
  bm = 2048
  assert batch % bm == 0
  return pl.pallas_call(
      _mm_body,
      grid=(batch // bm,),
      in_specs=[
          pl.BlockSpec((bm, dim), lambda i: (i, 0)),
          pl.BlockSpec((dim, odim), lambda i: (0, 0)),
          pl.BlockSpec((1, odim), lambda i: (0, 0)),
      ],
      out_specs=pl.BlockSpec((bm, odim), lambda i: (i, 0)),
      out_shape=jax.ShapeDtypeStruct((batch, odim), jnp.float32),
  )


def kernel(x, table, W, b):
  batch, hist = x.shape
  vocab, dim = table.shape
  odim = W.shape[1]
  x_flat = jnp.asarray(x, jnp.int32).reshape(batch * hist)
  x_flat = jnp.where(x_flat < _R, 2 * x_flat, 2 * x_flat - (2 * _S - 1))
  table_t, x_flat = lax.optimization_barrier((table.T, x_flat))
  table_lin = _build_pack(vocab, dim)(table_t, table_t).reshape(2 * _R, dim)
  pooled = _build_pool(batch, hist, dim)(x_flat, table_lin)
  return _build_linear(batch, dim, odim)(pooled, W, b.reshape(1, odim))


# speedup vs baseline: 1.6621x; 1.0887x over previous
import functools

import jax
import jax.numpy as jnp
from jax import lax
from jax.experimental import pallas as pl
from jax.experimental.pallas import tpu as pltpu
from jax.experimental.pallas import tpu_sc as plsc

LANES = 16


def _sc_worker_count() -> tuple[int, int]:
  try:
    info = plsc.get_sparse_core_info()
    return info.num_cores, info.num_subcores
  except Exception:
    return 2, 16


@functools.lru_cache(maxsize=None)
def _build_pool(batch: int, hist: int, dim: int):
  nc, ns = _sc_worker_count()
  nw = nc * ns
  assert batch % nw == 0
  bpw = batch // nw
  nbuf = 4
  chunk = 128
  assert bpw % chunk == 0 and chunk % nbuf == 0
  nch = bpw // chunk
  ngrp = chunk // nbuf - 1
  split = 128
  rest = hist - split
  assert 0 < rest <= 128 and hist % 8 == 0 and dim % LANES == 0
  nd = dim // LANES
  inv = jnp.float32(1.0 / hist)

  mesh = plsc.VectorSubcoreMesh(core_axis_name="c", subcore_axis_name="s")

  @functools.partial(
      pl.kernel,
      out_type=jax.ShapeDtypeStruct((batch, dim), jnp.float32),
      mesh=mesh,
      scratch_types=[
          pltpu.VMEM((chunk * hist,), jnp.int32),
          pltpu.VMEM((nbuf, hist, dim), jnp.float32),
          pltpu.VMEM((bpw, dim), jnp.float32),
          pltpu.SemaphoreType.DMA((nbuf,)),
      ],
      compiler_params=pltpu.CompilerParams(use_tc_tiling_on_sc=False),
  )
  def pool(x_hbm, table_hbm, out_hbm, idx_v, rows_v, acc_v, sem):
    wid = lax.axis_index("s") * nc + lax.axis_index("c")
    row0 = wid * bpw

    def issue(crow, slot):
      off = crow * hist
      pltpu.async_copy(
          table_hbm.at[idx_v.at[pl.ds(off, split)]],
          rows_v.at[slot, pl.ds(0, split)],
          sem.at[slot],
      )
      pltpu.async_copy(
          table_hbm.at[idx_v.at[pl.ds(off + split, rest)]],
          rows_v.at[slot, pl.ds(split, rest)],
          sem.at[slot],
      )

    def wait(slot):
      pltpu.make_async_copy(
          table_hbm.at[pl.ds(0, split)],
          rows_v.at[slot, pl.ds(0, split)],
          sem.at[slot],
      ).wait()
      pltpu.make_async_copy(
          table_hbm.at[pl.ds(0, rest)],
          rows_v.at[slot, pl.ds(split, rest)],
          sem.at[slot],
      ).wait()

    def reduce(slot, brow):
      r = rows_v.at[slot]

      def step(j, carry):
        return tuple(
            carry[d] + r[j, pl.ds(LANES * d, LANES)] for d in range(nd)
        )

      zeros = (jnp.zeros((LANES,), jnp.float32),) * nd
      acc = pl.loop(0, hist, init_carry=zeros, unroll=8)(step)
      for d in range(nd):
        acc_v[brow, pl.ds(LANES * d, LANES)] = acc[d] * inv

    def transform(_=None):
      def tstep(i):
        v = idx_v[pl.ds(i * LANES, LANES)]
        idx_v[pl.ds(i * LANES, LANES)] = jnp.where(
            v < _R, 2 * v, 2 * v - (2 * _S - 1))

      pl.loop(0, chunk * hist // LANES, unroll=8)(tstep)

    def chunk_body(c):
      base = c * chunk
      pltpu.sync_copy(
          x_hbm.at[pl.ds((row0 + base) * hist, chunk * hist)], idx_v
      )
      transform()
      for k in range(nbuf):
        issue(k, k)

      def grp(g):
        for k in range(nbuf):
          j = g * nbuf + k
          wait(k)
          reduce(k, base + j)
          issue(j + nbuf, k)

      pl.loop(0, ngrp)(grp)
      for k in range(nbuf):
        wait(k)
        reduce(k, base + (ngrp * nbuf + k))

    pl.loop(0, nch)(chunk_body)
    pltpu.sync_copy(acc_v, out_hbm.at[pl.ds(row0, bpw)])

  return pool


_PACK_B = 16128
_S = 499968
_R = 500032


def _pack_body(t1_ref, t2_ref, o_ref):
  eye = jnp.eye(64, dtype=jnp.float32)
  dn = (((0,), (0,)), ((), ()))
  o_ref[:, 0:64] = lax.dot_general(
      t1_ref[...], eye, dn, preferred_element_type=jnp.float32)
  o_ref[:, 64:128] = lax.dot_general(
      t2_ref[...], eye, dn, preferred_element_type=jnp.float32)


@functools.lru_cache(maxsize=None)
def _build_pack(vocab: int, dim: int):
  assert dim == 64 and vocab == 2 * _S + 64
  grid = (_R + _PACK_B - 1) // _PACK_B
  off = _S // _PACK_B
  return pl.pallas_call(
      _pack_body,
      grid=(grid,),
      in_specs=[
          pl.BlockSpec((dim, _PACK_B), lambda i: (0, i)),
          pl.BlockSpec((dim, _PACK_B), lambda i: (0, i + off)),
      ],
      out_specs=pl.BlockSpec((_PACK_B, 128), lambda i: (i, 0)),
      out_shape=jax.ShapeDtypeStruct((_R, 128), jnp.float32),
  )


def _mm_body(p_ref, w_ref, b_ref, o_ref):
  o_ref[...] = (
      jnp.dot(p_ref[...], w_ref[...], preferred_element_type=jnp.float32)
      + b_ref[...]
  )


@functools.lru_cache(maxsize=None)
def _build_linear(batch: int, dim: int, odim: int):
  bm = 2048
  assert batch % bm == 0
  return pl.pallas_call(
      _mm_body,
      grid=(batch // bm,),
      in_specs=[
          pl.BlockSpec((bm, dim), lambda i: (i, 0)),
          pl.BlockSpec((dim, odim), lambda i: (0, 0)),
          pl.BlockSpec((1, odim), lambda i: (0, 0)),
      ],
      out_specs=pl.BlockSpec((bm, odim), lambda i: (i, 0)),
      out_shape=jax.ShapeDtypeStruct((batch, odim), jnp.float32),
  )


def kernel(x, table, W, b):
  batch, hist = x.shape
  vocab, dim = table.shape
  odim = W.shape[1]
  x_flat = jnp.asarray(x, jnp.int32).reshape(batch * hist)
  table_t, x_flat = lax.optimization_barrier((table.T, x_flat))
  table_lin = _build_pack(vocab, dim)(table_t, table_t).reshape(2 * _R, dim)
  pooled = _build_pool(batch, hist, dim)(x_flat, table_lin)
  return _build_linear(batch, dim, odim)(pooled, W, b.reshape(1, odim))
